# baseline (speedup 1.0000x reference)
import jax
import jax.numpy as jnp
from jax.experimental import pallas as pl
from jax.experimental.pallas import tpu as pltpu

_NMS_RADIUS = 4
_MAX_KP = 1024


def _conv_body(xp_hbm, wcat_ref, bpa_ref, desc_ref, buf_ref, sem_ref):
    l = pl.program_id(0)
    w16 = wcat_ref[...].astype(jnp.bfloat16)
    bpa = bpa_ref[...]
    z = jnp.zeros((1, 64), jnp.float32)

    def dma(c, slot):
        return pltpu.make_async_copy(
            xp_hbm.at[l, pl.ds(c * 32, 34)], buf_ref.at[slot], sem_ref.at[slot])

    dma(0, 0).start()
    for c in range(4):
        if c + 1 < 4:
            dma(c + 1, (c + 1) % 2).start()
        dma(c, c % 2).wait()
        slot = c % 2

        def row(r, carry):
            xin = jnp.concatenate(
                [buf_ref[slot, r], buf_ref[slot, r + 1], buf_ref[slot, r + 2]],
                axis=1)
            o = jax.lax.dot_general(
                xin.astype(jnp.bfloat16), w16, (((1,), (0,)), ((), ())),
                preferred_element_type=jnp.float32)
            acc = (jnp.concatenate([z, o[0:255, 0:64]], axis=0)
                   + o[:, 64:128]
                   + jnp.concatenate([o[1:256, 128:192], z], axis=0)
                   + bpa)
            desc_ref[0, c * 32 + r] = jnp.maximum(acc, 0.0)
            return carry

        jax.lax.fori_loop(0, 32, row, 0)


def _conv_desc(xp, wcat, bpa):
    L = xp.shape[0]
    return pl.pallas_call(
        _conv_body,
        grid=(L,),
        in_specs=[
            pl.BlockSpec(memory_space=pl.ANY),
            pl.BlockSpec((384, 192), lambda l: (0, 0)),
            pl.BlockSpec((1, 64), lambda l: (0, 0)),
        ],
        out_specs=pl.BlockSpec((1, 128, 256, 64), lambda l: (l, 0, 0, 0)),
        out_shape=jax.ShapeDtypeStruct((L, 128, 256, 64), jnp.float32),
        scratch_shapes=[
            pltpu.VMEM((2, 34, 256, 128), jnp.float32),
            pltpu.SemaphoreType.DMA((2,)),
        ],
    )(xp, wcat, bpa)


def _maxpool9(a, neg):
    m = a
    for d in range(1, _NMS_RADIUS + 1):
        up = jnp.concatenate([a[d:], jnp.full((d, a.shape[1]), neg, a.dtype)], axis=0)
        dn = jnp.concatenate([jnp.full((d, a.shape[1]), neg, a.dtype), a[:-d]], axis=0)
        m = jnp.maximum(m, jnp.maximum(up, dn))
    a2 = m
    for d in range(1, _NMS_RADIUS + 1):
        lf = jnp.concatenate([a2[:, d:], jnp.full((a.shape[0], d), neg, a.dtype)], axis=1)
        rt = jnp.concatenate([jnp.full((a.shape[0], d), neg, a.dtype), a2[:, :-d]], axis=1)
        m = jnp.maximum(m, jnp.maximum(lf, rt))
    return m


def _nms_sort_body(s_ref, idx_ref):
    s = s_ref[...]
    neg = jnp.float32(-jnp.inf)
    per_agent = []
    for l in range(s.shape[0]):
        s2d = s[l]
        mm = s2d == _maxpool9(s2d, neg)
        for _ in range(2):
            supp = _maxpool9(mm.astype(jnp.float32), neg) > 0
            ss = jnp.where(supp, 0.0, s2d)
            nm = ss == _maxpool9(ss, neg)
            mm = mm | (nm & jnp.logical_not(supp))
        per_agent.append(jnp.where(mm, s2d, 0.0))
    keys = jnp.stack(per_agent)

    rr = jax.lax.broadcasted_iota(jnp.int32, keys.shape, 1)
    wl = jax.lax.broadcasted_iota(jnp.int32, keys.shape, 2)
    ee = rr * 256 + wl
    pay = ee

    n = 128 * 256
    k = 2
    while k <= n:
        j = k // 2
        while j >= 1:
            if j < 256:
                axis, amt, bit = 2, j, wl & j
            else:
                axis, amt, bit = 1, j // 256, rr & (j // 256)
            low = bit == 0
            ok = jnp.where(low, jnp.roll(keys, -amt, axis), jnp.roll(keys, amt, axis))
            op = jnp.where(low, jnp.roll(pay, -amt, axis), jnp.roll(pay, amt, axis))
            dirm = (ee & k) == 0
            c = (keys > ok) | ((keys == ok) & (pay < op))
            keep = jnp.logical_xor(dirm, jnp.logical_xor(low, c))
            keys = jnp.where(keep, keys, ok)
            pay = jnp.where(keep, pay, op)
            j //= 2
        k *= 2

    idx_ref[...] = pay[:, 0:4, :]


def _nms_topk(scores):
    L = scores.shape[0]
    return pl.pallas_call(
        _nms_sort_body,
        out_shape=jax.ShapeDtypeStruct((L, 4, 256), jnp.int32),
    )(scores)


def _affine_grid(theta, H, W):
    xs = (jnp.arange(W, dtype=theta.dtype) * 2.0 + 1.0) / W - 1.0
    ys = (jnp.arange(H, dtype=theta.dtype) * 2.0 + 1.0) / H - 1.0
    X, Y = jnp.meshgrid(xs, ys)
    coords = jnp.stack([X, Y, jnp.ones_like(X)], axis=-1)
    return jnp.einsum('nij,hwj->nhwi', theta, coords)


def _grid_sample(img, grid):
    N, C, H, W = img.shape
    ix = ((grid[..., 0] + 1.0) * W - 1.0) / 2.0
    iy = ((grid[..., 1] + 1.0) * H - 1.0) / 2.0
    x0 = jnp.floor(ix); y0 = jnp.floor(iy)
    x1 = x0 + 1.0; y1 = y0 + 1.0
    wx1 = ix - x0; wx0 = 1.0 - wx1
    wy1 = iy - y0; wy0 = 1.0 - wy1

    def gather(im, xi, yi):
        valid = ((xi >= 0) & (xi < W) & (yi >= 0) & (yi < H)).astype(im.dtype)
        xc = jnp.clip(xi, 0, W - 1).astype(jnp.int32)
        yc = jnp.clip(yi, 0, H - 1).astype(jnp.int32)
        return im[:, yc, xc] * valid[None]

    def per_n(im, x0n, y0n, x1n, y1n, wx0n, wx1n, wy0n, wy1n):
        v00 = gather(im, x0n, y0n)
        v01 = gather(im, x1n, y0n)
        v10 = gather(im, x0n, y1n)
        v11 = gather(im, x1n, y1n)
        return (v00 * (wx0n * wy0n)[None] + v01 * (wx1n * wy0n)[None]
                + v10 * (wx0n * wy1n)[None] + v11 * (wx1n * wy1n)[None])

    return jax.vmap(per_n)(img, x0, y0, x1, y1, wx0, wx1, wy0, wy1)


def _conv2d_ref(x, w, b, pad):
    out = jax.lax.conv_general_dilated(
        x, w, (1, 1), [(pad, pad), (pad, pad)],
        dimension_numbers=('NCHW', 'OIHW', 'NCHW'))
    return out + b[None, :, None, None]


def kernel(feats, convPa_w, convPa_b, convPb_w, convPb_b, fp_w, fp_b):
    L, C, H, W = feats.shape
    Cq = C // 2

    desc_x = jax.nn.relu(_conv2d_ref(feats, convPa_w, convPa_b, 1))
    scores = jax.nn.sigmoid(_conv2d_ref(desc_x, convPb_w, convPb_b, 0))[:, 0]

    idx = _nms_topk(scores).reshape(L, _MAX_KP)

    xt = jnp.transpose(feats, (0, 2, 3, 1))
    xp = jnp.pad(xt, ((0, 0), (1, 1), (0, 0), (0, 0)))
    wcat = jnp.transpose(convPa_w, (2, 1, 3, 0)).reshape(3 * C, 3 * Cq)
    desc = _conv_desc(xp, wcat, convPa_b.reshape(1, Cq))

    dflat = desc.reshape(L, H * W, Cq)
    dg = jnp.take_along_axis(dflat, idx[:, :, None], axis=1)
    nrm = jnp.sqrt(jnp.sum(dg * dg, axis=2, keepdims=True))
    x0 = (dg / jnp.maximum(nrm, 1e-12)).transpose(0, 2, 1)

    q = jnp.transpose(x0, (2, 0, 1))
    k = jnp.transpose(q, (0, 2, 1))
    sc = jnp.einsum('bnh,bhm->bnm', q, k) / (Cq ** 0.5)
    prob = jax.nn.softmax(sc, axis=-1)
    msg = jnp.einsum('bnm,bmh->bnh', prob, q)
    msg = jnp.transpose(msg, (1, 2, 0))

    d2 = x0 + (x0 + msg)
    d3 = jnp.einsum('oc,bcl->bol', fp_w[:, :, 0], d2) + fp_b[None, :, None]
    d3 = d3 - d3[0:1]
    mind = jnp.min(d3, axis=2)
    cos = jnp.cos(mind[:, 2]); sin = jnp.sin(mind[:, 2])
    row0 = jnp.stack([cos, -sin, mind[:, 0]], axis=-1)
    row1 = jnp.stack([sin, cos, mind[:, 1]], axis=-1)
    theta = jnp.stack([row0, row1], axis=1)
    grid = _affine_grid(theta, H, W)
    return _grid_sample(feats, grid)


# speedup vs baseline: 1.0962x; 1.0962x over previous
import jax
import jax.numpy as jnp
from jax.experimental import pallas as pl
from jax.experimental.pallas import tpu as pltpu

_NMS_RADIUS = 4
_MAX_KP = 1024


def _conv_body(xp_hbm, wcat_ref, bpa_ref, desc_ref, buf_ref, sem_ref):
    l = pl.program_id(0)
    w16 = wcat_ref[...].astype(jnp.bfloat16)
    bpa = bpa_ref[...]
    z = jnp.zeros((1, 64), jnp.float32)

    def dma(c, slot):
        return pltpu.make_async_copy(
            xp_hbm.at[l, pl.ds(c * 32, 34)], buf_ref.at[slot], sem_ref.at[slot])

    dma(0, 0).start()
    for c in range(4):
        if c + 1 < 4:
            dma(c + 1, (c + 1) % 2).start()
        dma(c, c % 2).wait()
        slot = c % 2

        def row(r, carry):
            xin = jnp.concatenate(
                [buf_ref[slot, r], buf_ref[slot, r + 1], buf_ref[slot, r + 2]],
                axis=1)
            o = jax.lax.dot_general(
                xin.astype(jnp.bfloat16), w16, (((1,), (0,)), ((), ())),
                preferred_element_type=jnp.float32)
            acc = (jnp.concatenate([z, o[0:255, 0:64]], axis=0)
                   + o[:, 64:128]
                   + jnp.concatenate([o[1:256, 128:192], z], axis=0)
                   + bpa)
            desc_ref[0, c * 32 + r] = jnp.maximum(acc, 0.0)
            return carry

        jax.lax.fori_loop(0, 32, row, 0)


def _conv_desc(xp, wcat, bpa):
    L = xp.shape[0]
    return pl.pallas_call(
        _conv_body,
        grid=(L,),
        in_specs=[
            pl.BlockSpec(memory_space=pl.ANY),
            pl.BlockSpec((384, 192), lambda l: (0, 0)),
            pl.BlockSpec((1, 64), lambda l: (0, 0)),
        ],
        out_specs=pl.BlockSpec((1, 128, 256, 64), lambda l: (l, 0, 0, 0)),
        out_shape=jax.ShapeDtypeStruct((L, 128, 256, 64), jnp.float32),
        scratch_shapes=[
            pltpu.VMEM((2, 34, 256, 128), jnp.float32),
            pltpu.SemaphoreType.DMA((2,)),
        ],
    )(xp, wcat, bpa)


def _maxpool9(a, neg):
    m = a
    for d in range(1, _NMS_RADIUS + 1):
        up = jnp.concatenate([a[d:], jnp.full((d, a.shape[1]), neg, a.dtype)], axis=0)
        dn = jnp.concatenate([jnp.full((d, a.shape[1]), neg, a.dtype), a[:-d]], axis=0)
        m = jnp.maximum(m, jnp.maximum(up, dn))
    a2 = m
    for d in range(1, _NMS_RADIUS + 1):
        lf = jnp.concatenate([a2[:, d:], jnp.full((a.shape[0], d), neg, a.dtype)], axis=1)
        rt = jnp.concatenate([jnp.full((a.shape[0], d), neg, a.dtype), a2[:, :-d]], axis=1)
        m = jnp.maximum(m, jnp.maximum(lf, rt))
    return m


def _nms_sort_body(s_ref, idx_ref):
    s = s_ref[...]
    neg = jnp.float32(-jnp.inf)
    per_agent = []
    for l in range(s.shape[0]):
        s2d = s[l]
        mm = s2d == _maxpool9(s2d, neg)
        for _ in range(2):
            supp = _maxpool9(mm.astype(jnp.float32), neg) > 0
            ss = jnp.where(supp, 0.0, s2d)
            nm = ss == _maxpool9(ss, neg)
            mm = mm | (nm & jnp.logical_not(supp))
        per_agent.append(jnp.where(mm, s2d, 0.0))
    keys = jnp.stack(per_agent)

    rr = jax.lax.broadcasted_iota(jnp.int32, keys.shape, 1)
    wl = jax.lax.broadcasted_iota(jnp.int32, keys.shape, 2)
    ee = rr * 256 + wl
    pay = ee

    n = 128 * 256
    k = 2
    while k <= n:
        j = k // 2
        while j >= 1:
            if j < 256:
                axis, amt, bit = 2, j, wl & j
            else:
                axis, amt, bit = 1, j // 256, rr & (j // 256)
            low = bit == 0
            ok = jnp.where(low, jnp.roll(keys, -amt, axis), jnp.roll(keys, amt, axis))
            op = jnp.where(low, jnp.roll(pay, -amt, axis), jnp.roll(pay, amt, axis))
            dirm = (ee & k) == 0
            c = (keys > ok) | ((keys == ok) & (pay < op))
            keep = jnp.logical_xor(dirm, jnp.logical_xor(low, c))
            keys = jnp.where(keep, keys, ok)
            pay = jnp.where(keep, pay, op)
            j //= 2
        k *= 2

    idx_ref[...] = pay[:, 0:4, :]


def _nms_topk(scores):
    L = scores.shape[0]
    return pl.pallas_call(
        _nms_sort_body,
        out_shape=jax.ShapeDtypeStruct((L, 4, 256), jnp.int32),
    )(scores)


def _affine_grid(theta, H, W):
    xs = (jnp.arange(W, dtype=theta.dtype) * 2.0 + 1.0) / W - 1.0
    ys = (jnp.arange(H, dtype=theta.dtype) * 2.0 + 1.0) / H - 1.0
    X, Y = jnp.meshgrid(xs, ys)
    coords = jnp.stack([X, Y, jnp.ones_like(X)], axis=-1)
    return jnp.einsum('nij,hwj->nhwi', theta, coords)


def _grid_sample(img, grid):
    N, C, H, W = img.shape
    ix = ((grid[..., 0] + 1.0) * W - 1.0) / 2.0
    iy = ((grid[..., 1] + 1.0) * H - 1.0) / 2.0
    x0 = jnp.floor(ix); y0 = jnp.floor(iy)
    x1 = x0 + 1.0; y1 = y0 + 1.0
    wx1 = ix - x0; wx0 = 1.0 - wx1
    wy1 = iy - y0; wy0 = 1.0 - wy1

    def gather(im, xi, yi):
        valid = ((xi >= 0) & (xi < W) & (yi >= 0) & (yi < H)).astype(im.dtype)
        xc = jnp.clip(xi, 0, W - 1).astype(jnp.int32)
        yc = jnp.clip(yi, 0, H - 1).astype(jnp.int32)
        return im[:, yc, xc] * valid[None]

    def per_n(im, x0n, y0n, x1n, y1n, wx0n, wx1n, wy0n, wy1n):
        v00 = gather(im, x0n, y0n)
        v01 = gather(im, x1n, y0n)
        v10 = gather(im, x0n, y1n)
        v11 = gather(im, x1n, y1n)
        return (v00 * (wx0n * wy0n)[None] + v01 * (wx1n * wy0n)[None]
                + v10 * (wx0n * wy1n)[None] + v11 * (wx1n * wy1n)[None])

    return jax.vmap(per_n)(img, x0, y0, x1, y1, wx0, wx1, wy0, wy1)


def _conv2d_ref(x, w, b, pad):
    out = jax.lax.conv_general_dilated(
        x, w, (1, 1), [(pad, pad), (pad, pad)],
        dimension_numbers=('NCHW', 'OIHW', 'NCHW'))
    return out + b[None, :, None, None]


def kernel(feats, convPa_w, convPa_b, convPb_w, convPb_b, fp_w, fp_b):
    L, C, H, W = feats.shape
    Cq = C // 2

    xt = jnp.transpose(feats, (0, 2, 3, 1))
    xp = jnp.pad(xt, ((0, 0), (1, 1), (0, 0), (0, 0)))
    wcat = jnp.transpose(convPa_w, (2, 1, 3, 0)).reshape(3 * C, 3 * Cq)
    desc = _conv_desc(xp, wcat, convPa_b.reshape(1, Cq))

    desc_nchw = jnp.transpose(desc, (0, 3, 1, 2))
    scores = jax.nn.sigmoid(_conv2d_ref(desc_nchw, convPb_w, convPb_b, 0))[:, 0]

    idx = _nms_topk(scores).reshape(L, _MAX_KP)

    dflat = desc.reshape(L, H * W, Cq)
    dg = jnp.take_along_axis(dflat, idx[:, :, None], axis=1)
    nrm = jnp.sqrt(jnp.sum(dg * dg, axis=2, keepdims=True))
    x0 = (dg / jnp.maximum(nrm, 1e-12)).transpose(0, 2, 1)

    q = jnp.transpose(x0, (2, 0, 1))
    k = jnp.transpose(q, (0, 2, 1))
    sc = jnp.einsum('bnh,bhm->bnm', q, k) / (Cq ** 0.5)
    prob = jax.nn.softmax(sc, axis=-1)
    msg = jnp.einsum('bnm,bmh->bnh', prob, q)
    msg = jnp.transpose(msg, (1, 2, 0))

    d2 = x0 + (x0 + msg)
    d3 = jnp.einsum('oc,bcl->bol', fp_w[:, :, 0], d2) + fp_b[None, :, None]
    d3 = d3 - d3[0:1]
    mind = jnp.min(d3, axis=2)
    cos = jnp.cos(mind[:, 2]); sin = jnp.sin(mind[:, 2])
    row0 = jnp.stack([cos, -sin, mind[:, 0]], axis=-1)
    row1 = jnp.stack([sin, cos, mind[:, 1]], axis=-1)
    theta = jnp.stack([row0, row1], axis=1)
    grid = _affine_grid(theta, H, W)
    return _grid_sample(feats, grid)
